# Initial kernel scaffold; baseline (speedup 1.0000x reference)
import jax
import jax.numpy as jnp
from jax.experimental import pallas as pl


def kernel(edge_attr, emb0, emb1, emb2):
    raise NotImplementedError("write your pallas kernel here")


# speedup vs baseline: 1.2285x; 1.2285x over previous
import functools

import jax
import jax.numpy as jnp
from jax import lax
from jax.experimental import pallas as pl
from jax.experimental.pallas import tpu as pltpu
from jax.experimental.pallas import tpu_sc as plsc

D = 64
V0, V1, V2 = 22, 6, 2
VC = V0 * V1 * V2
NC, NS, L = 2, 16, 16
NW = NC * NS
K = 80


def _table_body(e0_ref, e1_ref, e2_ref, out_ref):
    e0 = e0_ref[...]
    e1 = e1_ref[...]
    e2 = e2_ref[...]
    x0 = jnp.broadcast_to(e0[:, None, :], (V0, V1 * V2, D)).reshape(VC, D)
    x1 = jnp.broadcast_to(e1[None, :, None, :], (V0, V1, V2, D)).reshape(VC, D)
    x2 = jnp.broadcast_to(e2[None, :, :], (V0 * V1, V2, D)).reshape(VC, D)
    out_ref[...] = x0 + x1 + x2


def _build_table(e0, e1, e2):
    return pl.pallas_call(
        _table_body,
        out_shape=jax.ShapeDtypeStruct((VC, D), jnp.float32),
    )(e0, e1, e2)


@functools.lru_cache(maxsize=None)
def _make_gather(E):
    per_w = E // NW
    assert per_w * NW == E and per_w % K == 0
    nblk = per_w // K
    mesh = plsc.VectorSubcoreMesh(core_axis_name="c", subcore_axis_name="s")

    @functools.partial(
        pl.kernel,
        out_type=jax.ShapeDtypeStruct((E, D), jnp.float32),
        mesh=mesh,
        scratch_types=[
            pltpu.VMEM((K * 3,), jnp.int32),
            pltpu.VMEM((K,), jnp.int32),
            pltpu.VMEM((K, D), jnp.float32),
            pltpu.SemaphoreType.DMA,
        ],
        compiler_params=pltpu.CompilerParams(
            needs_layout_passes=False, use_tc_tiling_on_sc=False
        ),
    )
    def _gather(tbl_hbm, ea_hbm, out_hbm, ea_v, idx_v, rows_v, sem):
        wid = lax.axis_index("s") * NC + lax.axis_index("c")
        base = wid * per_w

        def blk(b, carry):
            eb = base + b * K
            pltpu.sync_copy(ea_hbm.at[pl.ds(eb * 3, K * 3)], ea_v)
            for t in range(K // L):
                pos = (lax.iota(jnp.int32, L) + (t * L)) * 3
                a0 = plsc.load_gather(ea_v, [pos])
                a1 = plsc.load_gather(ea_v, [pos + 1])
                a2 = plsc.load_gather(ea_v, [pos + 2])
                idx_v[pl.ds(t * L, L)] = a0 * (V1 * V2) + a1 * V2 + a2
            pltpu.async_copy(tbl_hbm.at[idx_v], rows_v, sem).wait()
            pltpu.sync_copy(rows_v, out_hbm.at[pl.ds(eb, K)])
            return carry

        lax.fori_loop(0, nblk, blk, 0)

    return _gather


@jax.jit
def kernel(edge_attr, emb0, emb1, emb2):
    E = edge_attr.shape[0]
    ea = edge_attr.astype(jnp.int32).reshape(E * 3)
    tbl = _build_table(emb0, emb1, emb2)
    return _make_gather(E)(tbl, ea)


# speedup vs baseline: 1.2310x; 1.0020x over previous
import functools

import jax
import jax.numpy as jnp
from jax import lax
from jax.experimental import pallas as pl
from jax.experimental.pallas import tpu as pltpu
from jax.experimental.pallas import tpu_sc as plsc

D = 64
V0, V1, V2 = 22, 6, 2
VC = V0 * V1 * V2
NC, NS, L = 2, 16, 16
NW = NC * NS
K = 80


def _table_body(e0_ref, e1_ref, e2_ref, out_ref):
    e0 = e0_ref[...]
    e1 = e1_ref[...]
    e2 = e2_ref[...]
    x0 = jnp.broadcast_to(e0[:, None, :], (V0, V1 * V2, D)).reshape(VC, D)
    x1 = jnp.broadcast_to(e1[None, :, None, :], (V0, V1, V2, D)).reshape(VC, D)
    x2 = jnp.broadcast_to(e2[None, :, :], (V0 * V1, V2, D)).reshape(VC, D)
    out_ref[...] = x0 + x1 + x2


def _build_table(e0, e1, e2):
    return pl.pallas_call(
        _table_body,
        out_shape=jax.ShapeDtypeStruct((VC, D), jnp.float32),
    )(e0, e1, e2)


G = 400
NBUF = 2
CH = 4000


@functools.lru_cache(maxsize=None)
def _make_gather(E):
    per_w = E // NW
    assert per_w * NW == E and per_w % G == 0 and per_w % CH == 0
    nblk = per_w // G
    nch = per_w // CH
    mesh = plsc.VectorSubcoreMesh(core_axis_name="c", subcore_axis_name="s")

    @functools.partial(
        pl.kernel,
        out_type=jax.ShapeDtypeStruct((E, D), jnp.float32),
        mesh=mesh,
        scratch_types=[
            pltpu.VMEM((CH * 3,), jnp.int32),
            pltpu.VMEM((per_w,), jnp.int32),
            pltpu.VMEM((G, D), jnp.float32),
            pltpu.VMEM((G, D), jnp.float32),
            pltpu.SemaphoreType.DMA,
            pltpu.SemaphoreType.DMA,
        ],
        compiler_params=pltpu.CompilerParams(
            needs_layout_passes=False, use_tc_tiling_on_sc=False
        ),
    )
    def _gather(tbl_hbm, ea_hbm, out_hbm, ea_v, idx_v, rows0, rows1, sg0, sg1):
        wid = lax.axis_index("s") * NC + lax.axis_index("c")
        base = wid * per_w

        lane3 = lax.iota(jnp.int32, L) * 3
        for c in range(nch):
            pltpu.sync_copy(ea_hbm.at[pl.ds((base + c * CH) * 3, CH * 3)], ea_v)

            def cbody(t, carry, c=c):
                pos = lane3 + t * (3 * L)
                a0 = plsc.load_gather(ea_v, [pos])
                a1 = plsc.load_gather(ea_v, [pos + 1])
                a2 = plsc.load_gather(ea_v, [pos + 2])
                idx_v[pl.ds(c * CH + t * L, L)] = a0 * (V1 * V2) + a1 * V2 + a2
                return carry

            lax.fori_loop(0, CH // L, cbody, 0)

        rows = [rows0, rows1]
        sems = [sg0, sg1]

        def issue(j, b):
            return pltpu.async_copy(
                tbl_hbm.at[idx_v.at[pl.ds(j * G, G)]], rows[b], sems[b]
            )

        hs = [issue(b, b) for b in range(NBUF)]
        for j in range(nblk):
            b = j % NBUF
            hs[b].wait()
            pltpu.sync_copy(rows[b], out_hbm.at[pl.ds(base + j * G, G)])
            if j + NBUF < nblk:
                hs[b] = issue(j + NBUF, b)

    return _gather


@jax.jit
def kernel(edge_attr, emb0, emb1, emb2):
    E = edge_attr.shape[0]
    ea = edge_attr.astype(jnp.int32).reshape(E * 3)
    tbl = _build_table(emb0, emb1, emb2)
    return _make_gather(E)(tbl, ea)


# speedup vs baseline: 21.2468x; 17.2605x over previous
import functools

import jax
import jax.numpy as jnp
from jax import lax
from jax.experimental import pallas as pl
from jax.experimental.pallas import tpu as pltpu
from jax.experimental.pallas import tpu_sc as plsc

D = 64
V0, V1, V2 = 22, 6, 2
VC = V0 * V1 * V2
NC, NS, L = 2, 16, 16
NW = NC * NS
BKJ = 2
NSLOT = 4


@functools.lru_cache(maxsize=None)
def _make_gather(E):
    NJ = E // 128
    assert NJ * 128 == E and NJ % BKJ == 0
    NB = NJ // BKJ
    base_nb = NB // NW
    extra = NB - base_nb * NW
    mesh = plsc.VectorSubcoreMesh(core_axis_name="c", subcore_axis_name="s")

    @functools.partial(
        pl.kernel,
        out_type=jax.ShapeDtypeStruct((8, NJ, 8, 128), jnp.float32),
        mesh=mesh,
        scratch_types=[
            pltpu.VMEM((V0 * D,), jnp.float32),
            pltpu.VMEM((V1 * D,), jnp.float32),
            pltpu.VMEM((V2 * D,), jnp.float32),
            pltpu.VMEM((D * VC,), jnp.float32),
            pltpu.VMEM((NSLOT, 3, BKJ * 128), jnp.int32),
            pltpu.VMEM((NSLOT, 8, BKJ, 8, 128), jnp.float32),
            pltpu.SemaphoreType.DMA,
            pltpu.SemaphoreType.DMA,
            pltpu.SemaphoreType.DMA,
            pltpu.SemaphoreType.DMA,
            pltpu.SemaphoreType.DMA,
            pltpu.SemaphoreType.DMA,
            pltpu.SemaphoreType.DMA,
            pltpu.SemaphoreType.DMA,
        ],
        compiler_params=pltpu.CompilerParams(
            needs_layout_passes=False, use_tc_tiling_on_sc=False
        ),
    )
    def _gather(e0_hbm, e1_hbm, e2_hbm, a0_hbm, a1_hbm, a2_hbm, out_hbm,
                e0v, e1v, e2v, tbl, av, ob,
                si0, si1, si2, si3, so0, so1, so2, so3):
        wid = lax.axis_index("s") * NC + lax.axis_index("c")
        lo = wid * base_nb + lax.min(wid, extra)
        nb = base_nb + jnp.where(wid < extra, 1, 0).astype(jnp.int32)
        sis = [si0, si1, si2, si3]
        sos = [so0, so1, so2, so3]
        a_hbms = [a0_hbm, a1_hbm, a2_hbm]
        iota = lax.iota(jnp.int32, L)
        iota_vc = iota * VC

        pltpu.sync_copy(e0_hbm, e0v)
        pltpu.sync_copy(e1_hbm, e1v)
        pltpu.sync_copy(e2_hbm, e2v)

        def bi(i, carry):
            def bj(j, carry):
                def bk(k, carry):
                    v = i * (V1 * V2) + j * V2 + k
                    for c in range(D // L):
                        row = (e0v[pl.ds(i * D + c * L, L)]
                               + e1v[pl.ds(j * D + c * L, L)]
                               + e2v[pl.ds(k * D + c * L, L)])
                        plsc.store_scatter(tbl, [iota_vc + (c * L * VC + v)], row)
                    return carry
                return lax.fori_loop(0, V2, bk, 0)
            return lax.fori_loop(0, V1, bj, 0)
        lax.fori_loop(0, V0, bi, 0)

        def issue_in(b, s):
            for r in range(3):
                pltpu.async_copy(
                    a_hbms[r].at[pl.ds((lo + b) * (BKJ * 128), BKJ * 128)],
                    av.at[s, r], sis[s],
                )

        def wait_in(s):
            for r in range(3):
                pltpu.make_async_copy(
                    a_hbms[r].at[pl.ds(0, BKJ * 128)], av.at[s, r], sis[s]
                ).wait()

        def issue_out(b, s):
            pltpu.async_copy(
                ob.at[s], out_hbm.at[:, pl.ds((lo + b) * BKJ, BKJ)], sos[s]
            )

        def wait_out(s):
            pltpu.make_async_copy(
                ob.at[s], out_hbm.at[:, pl.ds(0, BKJ)], sos[s]
            ).wait()

        for s in range(NSLOT):
            issue_in(s, s)

        def body(h, carry):
            for s in range(NSLOT):
                b = h * NSLOT + s

                @pl.when(b < nb)
                def _(b=b, s=s):
                    wait_in(s)

                    @pl.when(b >= NSLOT)
                    def _():
                        wait_out(s)

                    def grp(g, carry, s=s):
                        jj = g >> 3
                        c16 = (g & 7) * L
                        f = (av[s, 0, pl.ds(g * L, L)] * (V1 * V2)
                             + av[s, 1, pl.ds(g * L, L)] * V2
                             + av[s, 2, pl.ds(g * L, L)])
                        for d in range(D):
                            w = plsc.load_gather(tbl, [f + d * VC])
                            ob[s, d // 8, jj, d % 8, pl.ds(c16, L)] = w
                        return carry
                    lax.fori_loop(0, BKJ * 8, grp, 0)

                    issue_out(b, s)

                    @pl.when(b + NSLOT < nb)
                    def _():
                        issue_in(b + NSLOT, s)
            return carry

        lax.fori_loop(0, (nb + NSLOT - 1) // NSLOT, body, 0)

        for s in range(NSLOT):
            @pl.when(nb > s)
            def _(s=s):
                wait_out(s)

    return _gather


@jax.jit
def kernel(edge_attr, emb0, emb1, emb2):
    E = edge_attr.shape[0]
    ea = edge_attr.astype(jnp.int32)
    a0, a1, a2 = ea[:, 0], ea[:, 1], ea[:, 2]
    m = _make_gather(E)(
        emb0.reshape(V0 * D), emb1.reshape(V1 * D), emb2.reshape(V2 * D),
        a0, a1, a2,
    )
    return m.transpose(1, 3, 0, 2).reshape(E, D)


# speedup vs baseline: 47.7992x; 2.2497x over previous
import functools

import jax
import jax.numpy as jnp
from jax import lax
from jax.experimental import pallas as pl
from jax.experimental.pallas import tpu as pltpu
from jax.experimental.pallas import tpu_sc as plsc

D = 64
V0, V1, V2 = 22, 6, 2
VC = V0 * V1 * V2
NC, NS, L = 2, 16, 16
NW = NC * NS
BKJ = 2
NSLOT = 4


@functools.lru_cache(maxsize=None)
def _make_gather(E):
    NJ = E // 128
    assert NJ * 128 == E and NJ % BKJ == 0
    NB = NJ // BKJ
    base_nb = NB // NW
    extra = NB - base_nb * NW
    mesh = plsc.VectorSubcoreMesh(core_axis_name="c", subcore_axis_name="s")

    @functools.partial(
        pl.kernel,
        out_type=jax.ShapeDtypeStruct((8, NJ, 8, 128), jnp.float32),
        mesh=mesh,
        scratch_types=[
            pltpu.VMEM((V0 * D,), jnp.float32),
            pltpu.VMEM((V1 * D,), jnp.float32),
            pltpu.VMEM((V2 * D,), jnp.float32),
            pltpu.VMEM((D * VC,), jnp.float32),
            pltpu.VMEM((NSLOT, 3, BKJ * 128), jnp.int32),
            pltpu.VMEM((NSLOT, 8, BKJ, 8, 128), jnp.float32),
            pltpu.SemaphoreType.DMA,
            pltpu.SemaphoreType.DMA,
            pltpu.SemaphoreType.DMA,
            pltpu.SemaphoreType.DMA,
            pltpu.SemaphoreType.DMA,
            pltpu.SemaphoreType.DMA,
            pltpu.SemaphoreType.DMA,
            pltpu.SemaphoreType.DMA,
        ],
        compiler_params=pltpu.CompilerParams(
            needs_layout_passes=False, use_tc_tiling_on_sc=False
        ),
    )
    def _gather(e0_hbm, e1_hbm, e2_hbm, a0_hbm, a1_hbm, a2_hbm, out_hbm,
                e0v, e1v, e2v, tbl, av, ob,
                si0, si1, si2, si3, so0, so1, so2, so3):
        wid = lax.axis_index("s") * NC + lax.axis_index("c")
        lo = wid * base_nb + lax.min(wid, extra)
        nb = base_nb + jnp.where(wid < extra, 1, 0).astype(jnp.int32)
        sis = [si0, si1, si2, si3]
        sos = [so0, so1, so2, so3]
        a_hbms = [a0_hbm, a1_hbm, a2_hbm]
        iota = lax.iota(jnp.int32, L)
        iota_vc = iota * VC

        pltpu.sync_copy(e0_hbm, e0v)
        pltpu.sync_copy(e1_hbm, e1v)
        pltpu.sync_copy(e2_hbm, e2v)

        def bi(i, carry):
            def bj(j, carry):
                def bk(k, carry):
                    v = i * (V1 * V2) + j * V2 + k
                    for c in range(D // L):
                        row = (e0v[pl.ds(i * D + c * L, L)]
                               + e1v[pl.ds(j * D + c * L, L)]
                               + e2v[pl.ds(k * D + c * L, L)])
                        plsc.store_scatter(tbl, [iota_vc + (c * L * VC + v)], row)
                    return carry
                return lax.fori_loop(0, V2, bk, 0)
            return lax.fori_loop(0, V1, bj, 0)
        lax.fori_loop(0, V0, bi, 0)

        def issue_in(b, s):
            for r in range(3):
                pltpu.async_copy(
                    a_hbms[r].at[pl.ds((lo + b) * (BKJ * 128), BKJ * 128)],
                    av.at[s, r], sis[s],
                )

        def wait_in(s):
            for r in range(3):
                pltpu.make_async_copy(
                    a_hbms[r].at[pl.ds(0, BKJ * 128)], av.at[s, r], sis[s]
                ).wait()

        def issue_out(b, s):
            pltpu.async_copy(
                ob.at[s], out_hbm.at[:, pl.ds((lo + b) * BKJ, BKJ)], sos[s]
            )

        def wait_out(s):
            pltpu.make_async_copy(
                ob.at[s], out_hbm.at[:, pl.ds(0, BKJ)], sos[s]
            ).wait()

        for s in range(NSLOT):
            issue_in(s, s)

        def body(h, carry):
            for s in range(NSLOT):
                b = h * NSLOT + s

                @pl.when(b < nb)
                def _(b=b, s=s):
                    wait_in(s)

                    @pl.when(b >= NSLOT)
                    def _():
                        wait_out(s)

                    def grp(g, carry, s=s):
                        jj = g >> 3
                        c16 = (g & 7) * L
                        f = (av[s, 0, pl.ds(g * L, L)] * (V1 * V2)
                             + av[s, 1, pl.ds(g * L, L)] * V2
                             + av[s, 2, pl.ds(g * L, L)])
                        for d0 in range(0, D, 16):
                            ws = [plsc.load_gather(tbl, [f + d * VC])
                                  for d in range(d0, d0 + 16)]
                            for k, d in enumerate(range(d0, d0 + 16)):
                                ob[s, d // 8, jj, d % 8, pl.ds(c16, L)] = ws[k]
                        return carry
                    lax.fori_loop(0, BKJ * 8, grp, 0)

                    issue_out(b, s)

                    @pl.when(b + NSLOT < nb)
                    def _():
                        issue_in(b + NSLOT, s)
            return carry

        lax.fori_loop(0, (nb + NSLOT - 1) // NSLOT, body, 0)

        for s in range(NSLOT):
            @pl.when(nb > s)
            def _(s=s):
                wait_out(s)

    return _gather


@jax.jit
def kernel(edge_attr, emb0, emb1, emb2):
    E = edge_attr.shape[0]
    ea = edge_attr.astype(jnp.int32)
    a0, a1, a2 = ea[:, 0], ea[:, 1], ea[:, 2]
    m = _make_gather(E)(
        emb0.reshape(V0 * D), emb1.reshape(V1 * D), emb2.reshape(V2 * D),
        a0, a1, a2,
    )
    return m.transpose(1, 3, 0, 2).reshape(E, D)
